# Initial kernel scaffold; baseline (speedup 1.0000x reference)
import jax
import jax.numpy as jnp
from jax.experimental import pallas as pl


def kernel(x, edge_index, W1, b1, beta2):
    raise NotImplementedError("write your pallas kernel here")


# speedup vs baseline: 13.7137x; 13.7137x over previous
import functools

import jax
import jax.numpy as jnp
from jax import lax
from jax.experimental import pallas as pl
from jax.experimental.pallas import tpu as pltpu
from jax.experimental.pallas import tpu_sc as plsc

N = 10000
D = 128
E = 320000
NC = 2
NS = 16
NW = NC * NS
EPW = E // NW
C = 80
NCHUNK = EPW // C
N_ACC = 10240
RPT = N_ACC // NS
RB = 2000
EPS_NORM = 1e-12
EPS_DEN = 1e-16


def _prep_body(x_ref, w_ref, b_ref, h_ref, hn_ref, mult_ref, selfw_ref):
    xb = x_ref[...]
    h = jnp.maximum(
        lax.dot_general(xb, w_ref[...], (((1,), (0,)), ((), ())),
                        preferred_element_type=jnp.float32) + b_ref[...],
        0.0)
    ss = jnp.sum(h * h, axis=1, keepdims=True)
    cl = jnp.maximum(jnp.sqrt(ss), EPS_NORM)
    inv = 1.0 / cl
    h_ref[...] = h
    hn_ref[...] = h * inv
    mult_ref[...] = cl
    selfw_ref[...] = jnp.exp(ss * inv * inv)


def _tc_prep(x, w1t, b1r):
    return pl.pallas_call(
        _prep_body,
        grid=(N // RB,),
        in_specs=[
            pl.BlockSpec((RB, D), lambda i: (i, 0)),
            pl.BlockSpec((D, D), lambda i: (0, 0)),
            pl.BlockSpec((1, D), lambda i: (0, 0)),
        ],
        out_specs=[
            pl.BlockSpec((RB, D), lambda i: (i, 0)),
            pl.BlockSpec((RB, D), lambda i: (i, 0)),
            pl.BlockSpec((RB, 1), lambda i: (i, 0)),
            pl.BlockSpec((RB, 1), lambda i: (i, 0)),
        ],
        out_shape=[
            jax.ShapeDtypeStruct((N, D), jnp.float32),
            jax.ShapeDtypeStruct((N, D), jnp.float32),
            jax.ShapeDtypeStruct((N, 1), jnp.float32),
            jax.ShapeDtypeStruct((N, 1), jnp.float32),
        ],
    )(x, w1t, b1r)


def _combine_body(num_ref, den_ref, h_ref, selfw_ref, beta_ref,
                  o_ref, hn_ref, mult_ref, selfw2_ref):
    num = num_ref[0] + num_ref[1]
    den = jnp.sum(den_ref[...], axis=1, keepdims=True)
    sw = selfw_ref[...]
    h = h_ref[...]
    out = (num + sw * h) / (den + sw + EPS_DEN)
    ss = jnp.sum(out * out, axis=1, keepdims=True)
    cl = jnp.maximum(jnp.sqrt(ss), EPS_NORM)
    inv = 1.0 / cl
    o_ref[...] = out
    hn_ref[...] = out * inv
    mult_ref[...] = cl
    selfw2_ref[...] = jnp.exp(beta_ref[0, 0] * ss * inv * inv)


def _tc_combine(num, den_t, h, selfw, beta11):
    return pl.pallas_call(
        _combine_body,
        grid=(N // RB,),
        in_specs=[
            pl.BlockSpec((NC, RB, D), lambda i: (0, i, 0)),
            pl.BlockSpec((RB, NW), lambda i: (i, 0)),
            pl.BlockSpec((RB, D), lambda i: (i, 0)),
            pl.BlockSpec((RB, 1), lambda i: (i, 0)),
            pl.BlockSpec((1, 1), lambda i: (0, 0)),
        ],
        out_specs=[
            pl.BlockSpec((RB, D), lambda i: (i, 0)),
            pl.BlockSpec((RB, D), lambda i: (i, 0)),
            pl.BlockSpec((RB, 1), lambda i: (i, 0)),
            pl.BlockSpec((RB, 1), lambda i: (i, 0)),
        ],
        out_shape=[
            jax.ShapeDtypeStruct((N, D), jnp.float32),
            jax.ShapeDtypeStruct((N, D), jnp.float32),
            jax.ShapeDtypeStruct((N, 1), jnp.float32),
            jax.ShapeDtypeStruct((N, 1), jnp.float32),
        ],
    )(num, den_t, h, selfw, beta11)


def _sc_edge_body(hn_hbm, mult_hbm, src_hbm, dst_hbm, beta_hbm, zrows_hbm,
                  acc_hbm, den_hbm,
                  sidx, didx, srow, drow, multv, betav, denv, wtmp, ktmp,
                  numsp, sem_s, sem_d):
    cid = lax.axis_index("c")
    sid = lax.axis_index("s")
    wid = sid * NC + cid
    lane = lax.iota(jnp.int32, 16)
    z16 = jnp.zeros((16,), jnp.float32)

    pltpu.sync_copy(mult_hbm, multv)
    pltpu.sync_copy(beta_hbm, betav)
    pltpu.sync_copy(zrows_hbm, numsp.at[pl.ds(sid * RPT, RPT)])

    def zden(i, carry):
        denv[pl.ds(i * 16, 16)] = z16
        return carry
    lax.fori_loop(0, N // 16, zden, 0)
    plsc.subcore_barrier()
    beta = betav[...]

    def chunk(c, carry):
        base = wid * EPW + c * C
        pltpu.sync_copy(src_hbm.at[pl.ds(base, C)], sidx)
        pltpu.sync_copy(dst_hbm.at[pl.ds(base, C)], didx)
        cp_s = pltpu.async_copy(hn_hbm.at[sidx], srow, sem_s)
        cp_d = pltpu.async_copy(hn_hbm.at[didx], drow, sem_d)
        cp_s.wait()
        cp_d.wait()

        def group(g, carry2):
            off = g * 16
            dv = z16
            for e in range(16):
                acc = srow[off + e, pl.ds(0, 16)] * drow[off + e, pl.ds(0, 16)]
                for k in range(1, 8):
                    acc = acc + (srow[off + e, pl.ds(k * 16, 16)]
                                 * drow[off + e, pl.ds(k * 16, 16)])
                dot = jnp.sum(acc)
                dv = jnp.where(lane == e, dot, dv)
            si = sidx[pl.ds(off, 16)]
            di = didx[pl.ds(off, 16)]
            w = jnp.exp(beta * dv)
            w = jnp.where(si == di, 0.0, w)
            ws = w * plsc.load_gather(multv, [si])
            for e in range(16):
                wse = ws[e]
                for k in range(8):
                    drow[off + e, pl.ds(k * 16, 16)] = (
                        srow[off + e, pl.ds(k * 16, 16)] * wse)
            sk, sw = plsc.sort_key_val(di, w)
            csum = plsc.cumsum(sw)
            ktmp[...] = sk
            key_next = plsc.load_gather(ktmp, [jnp.minimum(lane + 1, 15)])
            key_prev = plsc.load_gather(ktmp, [jnp.maximum(lane - 1, 0)])
            is_end = (lane == 15) | (sk != key_next)
            is_start = (lane == 0) | (sk != key_prev)
            sstart = plsc.cummax(jnp.where(is_start, lane, 0))
            wtmp[...] = csum
            seg_base = jnp.where(
                sstart == 0, 0.0,
                plsc.load_gather(wtmp, [jnp.maximum(sstart - 1, 0)]))
            plsc.addupdate_scatter(denv, [sk], csum - seg_base, mask=is_end)
            return carry2

        lax.fori_loop(0, C // 16, group, 0)
        pltpu.sync_copy(drow, numsp.at[didx], add=True)
        return carry

    lax.fori_loop(0, NCHUNK, chunk, 0)
    plsc.subcore_barrier()
    pltpu.sync_copy(numsp.at[pl.ds(sid * RPT, RPT)],
                    acc_hbm.at[cid, pl.ds(sid * RPT, RPT)])
    pltpu.sync_copy(denv, den_hbm.at[wid])


def _sc_edge(hn, mult_flat, src, dst, beta16, zrows):
    mesh = plsc.VectorSubcoreMesh(core_axis_name="c", subcore_axis_name="s")
    f = functools.partial(
        pl.kernel,
        out_type=(jax.ShapeDtypeStruct((NC, N_ACC, D), jnp.float32),
                  jax.ShapeDtypeStruct((NW, N), jnp.float32)),
        mesh=mesh,
        compiler_params=pltpu.CompilerParams(needs_layout_passes=False),
        scratch_types=[
            pltpu.VMEM((C,), jnp.int32),
            pltpu.VMEM((C,), jnp.int32),
            pltpu.VMEM((C, D), jnp.float32),
            pltpu.VMEM((C, D), jnp.float32),
            pltpu.VMEM((N,), jnp.float32),
            pltpu.VMEM((16,), jnp.float32),
            pltpu.VMEM((N,), jnp.float32),
            pltpu.VMEM((16,), jnp.float32),
            pltpu.VMEM((16,), jnp.int32),
            pltpu.VMEM_SHARED((N_ACC, D), jnp.float32),
            pltpu.SemaphoreType.DMA,
            pltpu.SemaphoreType.DMA,
        ],
    )(_sc_edge_body)
    return f(hn, mult_flat, src, dst, beta16, zrows)


def kernel(x, edge_index, W1, b1, beta2):
    w1t = W1.T
    b1r = b1.reshape(1, D)
    beta11 = beta2.reshape(1, 1).astype(jnp.float32)
    src = edge_index[0]
    dst = edge_index[1]
    zrows = jnp.zeros((RPT, D), jnp.float32)
    beta1v = jnp.ones((16,), jnp.float32)
    beta2v = jnp.broadcast_to(beta2, (16,)).astype(jnp.float32)

    h1, hn1, mult1, selfw1 = _tc_prep(x, w1t, b1r)
    num1, den1 = _sc_edge(hn1, mult1.reshape(N), src, dst, beta1v, zrows)
    h2, hn2, mult2, selfw2 = _tc_combine(num1, den1.T, h1, selfw1, beta11)
    num2, den2 = _sc_edge(hn2, mult2.reshape(N), src, dst, beta2v, zrows)
    out, _, _, _ = _tc_combine(num2, den2.T, h2, selfw2, beta11)
    return out


# speedup vs baseline: 17.6761x; 1.2889x over previous
import functools

import jax
import jax.numpy as jnp
from jax import lax
from jax.experimental import pallas as pl
from jax.experimental.pallas import tpu as pltpu
from jax.experimental.pallas import tpu_sc as plsc

N = 10000
D = 128
E = 320000
NC = 2
NS = 16
NW = NC * NS
EPW = E // NW
C = 64
TAIL = 16
NFULL = (EPW - TAIL) // C
NPAIR = NFULL // 2
ZR = 2000
RB = 2000
EPS_NORM = 1e-12
EPS_DEN = 1e-16


def _prep_body(x_ref, w_ref, b_ref, h_ref, hn_ref, mult_ref, selfw_ref):
    xb = x_ref[...]
    h = jnp.maximum(
        lax.dot_general(xb, w_ref[...], (((1,), (0,)), ((), ())),
                        preferred_element_type=jnp.float32) + b_ref[...],
        0.0)
    ss = jnp.sum(h * h, axis=1, keepdims=True)
    cl = jnp.maximum(jnp.sqrt(ss), EPS_NORM)
    inv = 1.0 / cl
    h_ref[...] = h
    hn_ref[...] = h * inv
    mult_ref[...] = cl
    selfw_ref[...] = jnp.exp(ss * inv * inv)


def _tc_prep(x, w1t, b1r):
    return pl.pallas_call(
        _prep_body,
        grid=(N // RB,),
        in_specs=[
            pl.BlockSpec((RB, D), lambda i: (i, 0)),
            pl.BlockSpec((D, D), lambda i: (0, 0)),
            pl.BlockSpec((1, D), lambda i: (0, 0)),
        ],
        out_specs=[
            pl.BlockSpec((RB, D), lambda i: (i, 0)),
            pl.BlockSpec((RB, D), lambda i: (i, 0)),
            pl.BlockSpec((RB, 1), lambda i: (i, 0)),
            pl.BlockSpec((RB, 1), lambda i: (i, 0)),
        ],
        out_shape=[
            jax.ShapeDtypeStruct((N, D), jnp.float32),
            jax.ShapeDtypeStruct((N, D), jnp.float32),
            jax.ShapeDtypeStruct((N, 1), jnp.float32),
            jax.ShapeDtypeStruct((N, 1), jnp.float32),
        ],
    )(x, w1t, b1r)


def _combine_body(num_ref, den_ref, h_ref, selfw_ref, beta_ref,
                  o_ref, hn_ref, mult_ref, selfw2_ref):
    num = num_ref[0] + num_ref[1]
    den = jnp.sum(den_ref[...], axis=1, keepdims=True)
    sw = selfw_ref[...]
    h = h_ref[...]
    out = (num + sw * h) / (den + sw + EPS_DEN)
    ss = jnp.sum(out * out, axis=1, keepdims=True)
    cl = jnp.maximum(jnp.sqrt(ss), EPS_NORM)
    inv = 1.0 / cl
    o_ref[...] = out
    hn_ref[...] = out * inv
    mult_ref[...] = cl
    selfw2_ref[...] = jnp.exp(beta_ref[0, 0] * ss * inv * inv)


def _tc_combine(num, den_t, h, selfw, beta11):
    return pl.pallas_call(
        _combine_body,
        grid=(N // RB,),
        in_specs=[
            pl.BlockSpec((NC, RB, D), lambda i: (0, i, 0)),
            pl.BlockSpec((RB, NW), lambda i: (i, 0)),
            pl.BlockSpec((RB, D), lambda i: (i, 0)),
            pl.BlockSpec((RB, 1), lambda i: (i, 0)),
            pl.BlockSpec((1, 1), lambda i: (0, 0)),
        ],
        out_specs=[
            pl.BlockSpec((RB, D), lambda i: (i, 0)),
            pl.BlockSpec((RB, D), lambda i: (i, 0)),
            pl.BlockSpec((RB, 1), lambda i: (i, 0)),
            pl.BlockSpec((RB, 1), lambda i: (i, 0)),
        ],
        out_shape=[
            jax.ShapeDtypeStruct((N, D), jnp.float32),
            jax.ShapeDtypeStruct((N, D), jnp.float32),
            jax.ShapeDtypeStruct((N, 1), jnp.float32),
            jax.ShapeDtypeStruct((N, 1), jnp.float32),
        ],
    )(num, den_t, h, selfw, beta11)


def _sc_edge_body(hn_hbm, mult_hbm, src_hbm, dst_hbm, beta_hbm, zrows_hbm,
                  acc_hbm, den_hbm,
                  sidxA, didxA, sidxB, didxB, srowA, drowA, srowB, drowB,
                  multA, multB, betav, denv, wtmp, ktmp,
                  numsp, semA, semB):
    cid = lax.axis_index("c")
    sid = lax.axis_index("s")
    wid = sid * NC + cid
    lane = lax.iota(jnp.int32, 16)
    z16 = jnp.zeros((16,), jnp.float32)
    zi16 = jnp.zeros((16,), jnp.int32)
    wbase = wid * EPW

    pltpu.sync_copy(beta_hbm, betav)

    @pl.when(sid < N // ZR)
    def _zero_acc():
        pltpu.sync_copy(zrows_hbm, numsp.at[pl.ds(sid * ZR, ZR)])

    def zden(i, carry):
        denv[pl.ds(i * 16, 16)] = z16
        return carry
    lax.fori_loop(0, N // 16, zden, 0)
    plsc.subcore_barrier()
    beta = betav[...]

    def do_group(srow, drow, sidx, didx, multb, off):
        dv = z16
        for e in range(16):
            acc = srow[off + e, pl.ds(0, 16)] * drow[off + e, pl.ds(0, 16)]
            for k in range(1, 8):
                acc = acc + (srow[off + e, pl.ds(k * 16, 16)]
                             * drow[off + e, pl.ds(k * 16, 16)])
            dot = jnp.sum(acc)
            dv = jnp.where(lane == e, dot, dv)
        si = sidx[pl.ds(off, 16)]
        di = didx[pl.ds(off, 16)]
        w = jnp.exp(beta * dv)
        w = jnp.where(si == di, 0.0, w)
        ws = w * multb[pl.ds(off, 16)]
        for e in range(16):
            wse = ws[e]
            for k in range(8):
                drow[off + e, pl.ds(k * 16, 16)] = (
                    srow[off + e, pl.ds(k * 16, 16)] * wse)
        sk, sw = plsc.sort_key_val(di, w)
        csum = plsc.cumsum(sw)
        ktmp[...] = sk
        key_next = plsc.load_gather(ktmp, [jnp.minimum(lane + 1, 15)])
        key_prev = plsc.load_gather(ktmp, [jnp.maximum(lane - 1, 0)])
        is_end = (lane == 15) | (sk != key_next)
        is_start = (lane == 0) | (sk != key_prev)
        sstart = plsc.cummax(jnp.where(is_start, lane, 0))
        wtmp[...] = csum
        seg_base = jnp.where(
            sstart == 0, 0.0,
            plsc.load_gather(wtmp, [jnp.maximum(sstart - 1, 0)]))
        plsc.addupdate_scatter(denv, [sk], csum - seg_base, mask=is_end)

    def load_idx_and_fire(base, sidx, didx, multb, srow, drow, sem):
        pltpu.sync_copy(src_hbm.at[pl.ds(base, C)], sidx)
        pltpu.sync_copy(dst_hbm.at[pl.ds(base, C)], didx)
        pltpu.async_copy(hn_hbm.at[sidx], srow, sem)
        pltpu.async_copy(hn_hbm.at[didx], drow, sem)
        pltpu.async_copy(mult_hbm.at[sidx], multb, sem)

    def wait_set(srow, drow, multb, sem):
        pltpu.make_async_copy(hn_hbm.at[pl.ds(0, C)], srow, sem).wait()
        pltpu.make_async_copy(hn_hbm.at[pl.ds(0, C)], drow, sem).wait()
        pltpu.make_async_copy(mult_hbm.at[pl.ds(0, C)], multb, sem).wait()

    def compute_chunk(sidx, didx, srow, drow, multb):
        def grp(g, carry):
            do_group(srow, drow, sidx, didx, multb, g * 16)
            return carry
        lax.fori_loop(0, C // 16, grp, 0)
        pltpu.sync_copy(drow, numsp.at[didx], add=True)

    load_idx_and_fire(wbase, sidxA, didxA, multA, srowA, drowA, semA)
    wait_set(srowA, drowA, multA, semA)
    do_group(srowA, drowA, sidxA, didxA, multA, 0)
    for r in range(TAIL, C):
        for k in range(8):
            drowA[r, pl.ds(k * 16, 16)] = z16
    for r in range(TAIL, C, 16):
        didxA[pl.ds(r, 16)] = zi16
    pltpu.sync_copy(drowA, numsp.at[didxA], add=True)

    load_idx_and_fire(wbase + TAIL, sidxA, didxA, multA, srowA, drowA, semA)

    def pair(i, carry):
        wait_set(srowA, drowA, multA, semA)
        load_idx_and_fire(wbase + TAIL + (2 * i + 1) * C,
                          sidxB, didxB, multB, srowB, drowB, semB)
        compute_chunk(sidxA, didxA, srowA, drowA, multA)

        @pl.when(i < NPAIR - 1)
        def _prefetch_a():
            load_idx_and_fire(wbase + TAIL + (2 * i + 2) * C,
                              sidxA, didxA, multA, srowA, drowA, semA)

        wait_set(srowB, drowB, multB, semB)
        compute_chunk(sidxB, didxB, srowB, drowB, multB)
        return carry

    lax.fori_loop(0, NPAIR, pair, 0)

    plsc.subcore_barrier()

    @pl.when(sid < N // ZR)
    def _write_acc():
        pltpu.sync_copy(numsp.at[pl.ds(sid * ZR, ZR)],
                        acc_hbm.at[cid, pl.ds(sid * ZR, ZR)])

    pltpu.sync_copy(denv, den_hbm.at[wid])


def _sc_edge(hn, mult_flat, src, dst, beta16, zrows):
    mesh = plsc.VectorSubcoreMesh(core_axis_name="c", subcore_axis_name="s")
    f = functools.partial(
        pl.kernel,
        out_type=(jax.ShapeDtypeStruct((NC, N, D), jnp.float32),
                  jax.ShapeDtypeStruct((NW, N), jnp.float32)),
        mesh=mesh,
        compiler_params=pltpu.CompilerParams(needs_layout_passes=False),
        scratch_types=[
            pltpu.VMEM((C,), jnp.int32),
            pltpu.VMEM((C,), jnp.int32),
            pltpu.VMEM((C,), jnp.int32),
            pltpu.VMEM((C,), jnp.int32),
            pltpu.VMEM((C, D), jnp.float32),
            pltpu.VMEM((C, D), jnp.float32),
            pltpu.VMEM((C, D), jnp.float32),
            pltpu.VMEM((C, D), jnp.float32),
            pltpu.VMEM((C,), jnp.float32),
            pltpu.VMEM((C,), jnp.float32),
            pltpu.VMEM((16,), jnp.float32),
            pltpu.VMEM((N,), jnp.float32),
            pltpu.VMEM((16,), jnp.float32),
            pltpu.VMEM((16,), jnp.int32),
            pltpu.VMEM_SHARED((N, D), jnp.float32),
            pltpu.SemaphoreType.DMA,
            pltpu.SemaphoreType.DMA,
        ],
    )(_sc_edge_body)
    return f(hn, mult_flat, src, dst, beta16, zrows)


def kernel(x, edge_index, W1, b1, beta2):
    w1t = W1.T
    b1r = b1.reshape(1, D)
    beta11 = beta2.reshape(1, 1).astype(jnp.float32)
    src = edge_index[0]
    dst = edge_index[1]
    zrows = jnp.zeros((ZR, D), jnp.float32)
    beta1v = jnp.ones((16,), jnp.float32)
    beta2v = jnp.broadcast_to(beta2, (16,)).astype(jnp.float32)

    h1, hn1, mult1, selfw1 = _tc_prep(x, w1t, b1r)
    num1, den1 = _sc_edge(hn1, mult1.reshape(N), src, dst, beta1v, zrows)
    h2, hn2, mult2, selfw2 = _tc_combine(num1, den1.T, h1, selfw1, beta11)
    num2, den2 = _sc_edge(hn2, mult2.reshape(N), src, dst, beta2v, zrows)
    out, _, _, _ = _tc_combine(num2, den2.T, h2, selfw2, beta11)
    return out


# speedup vs baseline: 18.7549x; 1.0610x over previous
import functools

import jax
import jax.numpy as jnp
from jax import lax
from jax.experimental import pallas as pl
from jax.experimental.pallas import tpu as pltpu
from jax.experimental.pallas import tpu_sc as plsc

N = 10000
D = 128
E = 320000
NC = 2
NS = 16
NW = NC * NS
EPW = E // NW
C = 64
TAIL = 16
NFULL = (EPW - TAIL) // C
NPAIR = NFULL // 2
ZR = 2000
RB = 2000
EPS_NORM = 1e-12
EPS_DEN = 1e-16


def _prep_body(x_ref, w_ref, b_ref, h_ref, hn_ref, mult_ref, selfw_ref):
    xb = x_ref[...]
    h = jnp.maximum(
        lax.dot_general(xb, w_ref[...], (((1,), (0,)), ((), ())),
                        preferred_element_type=jnp.float32) + b_ref[...],
        0.0)
    ss = jnp.sum(h * h, axis=1, keepdims=True)
    cl = jnp.maximum(jnp.sqrt(ss), EPS_NORM)
    inv = 1.0 / cl
    h_ref[...] = h
    hn_ref[...] = h * inv
    mult_ref[...] = cl
    selfw_ref[...] = jnp.exp(ss * inv * inv)


def _tc_prep(x, w1t, b1r):
    return pl.pallas_call(
        _prep_body,
        grid=(N // RB,),
        in_specs=[
            pl.BlockSpec((RB, D), lambda i: (i, 0)),
            pl.BlockSpec((D, D), lambda i: (0, 0)),
            pl.BlockSpec((1, D), lambda i: (0, 0)),
        ],
        out_specs=[
            pl.BlockSpec((RB, D), lambda i: (i, 0)),
            pl.BlockSpec((RB, D), lambda i: (i, 0)),
            pl.BlockSpec((RB, 1), lambda i: (i, 0)),
            pl.BlockSpec((RB, 1), lambda i: (i, 0)),
        ],
        out_shape=[
            jax.ShapeDtypeStruct((N, D), jnp.float32),
            jax.ShapeDtypeStruct((N, D), jnp.float32),
            jax.ShapeDtypeStruct((N, 1), jnp.float32),
            jax.ShapeDtypeStruct((N, 1), jnp.float32),
        ],
    )(x, w1t, b1r)


def _combine_body(num_ref, den_ref, h_ref, selfw_ref, beta_ref,
                  o_ref, hn_ref, mult_ref, selfw2_ref):
    num = num_ref[0] + num_ref[1]
    den = jnp.sum(den_ref[...], axis=1, keepdims=True)
    sw = selfw_ref[...]
    h = h_ref[...]
    out = (num + sw * h) / (den + sw + EPS_DEN)
    ss = jnp.sum(out * out, axis=1, keepdims=True)
    cl = jnp.maximum(jnp.sqrt(ss), EPS_NORM)
    inv = 1.0 / cl
    o_ref[...] = out
    hn_ref[...] = out * inv
    mult_ref[...] = cl
    selfw2_ref[...] = jnp.exp(beta_ref[0, 0] * ss * inv * inv)


def _tc_combine(num, den_t, h, selfw, beta11):
    return pl.pallas_call(
        _combine_body,
        grid=(N // RB,),
        in_specs=[
            pl.BlockSpec((NC, RB, D), lambda i: (0, i, 0)),
            pl.BlockSpec((RB, NW), lambda i: (i, 0)),
            pl.BlockSpec((RB, D), lambda i: (i, 0)),
            pl.BlockSpec((RB, 1), lambda i: (i, 0)),
            pl.BlockSpec((1, 1), lambda i: (0, 0)),
        ],
        out_specs=[
            pl.BlockSpec((RB, D), lambda i: (i, 0)),
            pl.BlockSpec((RB, D), lambda i: (i, 0)),
            pl.BlockSpec((RB, 1), lambda i: (i, 0)),
            pl.BlockSpec((RB, 1), lambda i: (i, 0)),
        ],
        out_shape=[
            jax.ShapeDtypeStruct((N, D), jnp.float32),
            jax.ShapeDtypeStruct((N, D), jnp.float32),
            jax.ShapeDtypeStruct((N, 1), jnp.float32),
            jax.ShapeDtypeStruct((N, 1), jnp.float32),
        ],
    )(num, den_t, h, selfw, beta11)


def _sc_edge_body(hn_hbm, mult_hbm, src_hbm, dst_hbm, beta_hbm, zrows_hbm,
                  acc_hbm, den_hbm,
                  sidxA, didxA, sidxB, didxB, srowA, drowA, srowB, drowB,
                  multA, multB, betav, denv, wtmp, ktmp,
                  numsp, semA, semB):
    cid = lax.axis_index("c")
    sid = lax.axis_index("s")
    wid = sid * NC + cid
    lane = lax.iota(jnp.int32, 16)
    z16 = jnp.zeros((16,), jnp.float32)
    zi16 = jnp.zeros((16,), jnp.int32)
    wbase = wid * EPW

    pltpu.sync_copy(beta_hbm, betav)

    @pl.when(sid < N // ZR)
    def _zero_acc():
        pltpu.sync_copy(zrows_hbm, numsp.at[pl.ds(sid * ZR, ZR)])

    def zden(i, carry):
        denv[pl.ds(i * 16, 16)] = z16
        return carry
    lax.fori_loop(0, N // 16, zden, 0)
    plsc.subcore_barrier()
    beta = betav[...]

    def do_group(srow, drow, sidx, didx, multb, off):
        si = sidx[pl.ds(off, 16)]
        di = didx[pl.ds(off, 16)]
        mg = multb[pl.ds(off, 16)]
        dv = z16
        for e in range(16):
            sc = [srow[off + e, pl.ds(k * 16, 16)] for k in range(8)]
            acc = sc[0] * drow[off + e, pl.ds(0, 16)]
            for k in range(1, 8):
                acc = acc + sc[k] * drow[off + e, pl.ds(k * 16, 16)]
            dot = jnp.sum(acc)
            dv = jnp.where(lane == e, dot, dv)
            wev = jnp.exp(beta * dot)
            wse = jnp.where(si[e] == di[e], 0.0, wev[0] * mg[e])
            for k in range(8):
                drow[off + e, pl.ds(k * 16, 16)] = sc[k] * wse
        w = jnp.exp(beta * dv)
        w = jnp.where(si == di, 0.0, w)
        sk, sw = plsc.sort_key_val(di, w)
        csum = plsc.cumsum(sw)
        ktmp[...] = sk
        key_next = plsc.load_gather(ktmp, [jnp.minimum(lane + 1, 15)])
        key_prev = plsc.load_gather(ktmp, [jnp.maximum(lane - 1, 0)])
        is_end = (lane == 15) | (sk != key_next)
        is_start = (lane == 0) | (sk != key_prev)
        sstart = plsc.cummax(jnp.where(is_start, lane, 0))
        wtmp[...] = csum
        seg_base = jnp.where(
            sstart == 0, 0.0,
            plsc.load_gather(wtmp, [jnp.maximum(sstart - 1, 0)]))
        plsc.addupdate_scatter(denv, [sk], csum - seg_base, mask=is_end)

    def load_idx_and_fire(base, sidx, didx, multb, srow, drow, sem):
        pltpu.sync_copy(src_hbm.at[pl.ds(base, C)], sidx)
        pltpu.sync_copy(dst_hbm.at[pl.ds(base, C)], didx)
        pltpu.async_copy(hn_hbm.at[sidx], srow, sem)
        pltpu.async_copy(hn_hbm.at[didx], drow, sem)
        pltpu.async_copy(mult_hbm.at[sidx], multb, sem)

    def wait_set(srow, drow, multb, sem):
        pltpu.make_async_copy(hn_hbm.at[pl.ds(0, C)], srow, sem).wait()
        pltpu.make_async_copy(hn_hbm.at[pl.ds(0, C)], drow, sem).wait()
        pltpu.make_async_copy(mult_hbm.at[pl.ds(0, C)], multb, sem).wait()

    def compute_chunk(sidx, didx, srow, drow, multb):
        def grp(g, carry):
            do_group(srow, drow, sidx, didx, multb, g * 16)
            return carry
        lax.fori_loop(0, C // 16, grp, 0)
        pltpu.sync_copy(drow, numsp.at[didx], add=True)

    load_idx_and_fire(wbase, sidxA, didxA, multA, srowA, drowA, semA)
    wait_set(srowA, drowA, multA, semA)
    do_group(srowA, drowA, sidxA, didxA, multA, 0)
    for r in range(TAIL, C):
        for k in range(8):
            drowA[r, pl.ds(k * 16, 16)] = z16
    for r in range(TAIL, C, 16):
        didxA[pl.ds(r, 16)] = zi16
    pltpu.sync_copy(drowA, numsp.at[didxA], add=True)

    load_idx_and_fire(wbase + TAIL, sidxA, didxA, multA, srowA, drowA, semA)

    def pair(i, carry):
        wait_set(srowA, drowA, multA, semA)
        load_idx_and_fire(wbase + TAIL + (2 * i + 1) * C,
                          sidxB, didxB, multB, srowB, drowB, semB)
        compute_chunk(sidxA, didxA, srowA, drowA, multA)

        @pl.when(i < NPAIR - 1)
        def _prefetch_a():
            load_idx_and_fire(wbase + TAIL + (2 * i + 2) * C,
                              sidxA, didxA, multA, srowA, drowA, semA)

        wait_set(srowB, drowB, multB, semB)
        compute_chunk(sidxB, didxB, srowB, drowB, multB)
        return carry

    lax.fori_loop(0, NPAIR, pair, 0)

    plsc.subcore_barrier()

    @pl.when(sid < N // ZR)
    def _write_acc():
        pltpu.sync_copy(numsp.at[pl.ds(sid * ZR, ZR)],
                        acc_hbm.at[cid, pl.ds(sid * ZR, ZR)])

    pltpu.sync_copy(denv, den_hbm.at[wid])


def _sc_edge(hn, mult_flat, src, dst, beta16, zrows):
    mesh = plsc.VectorSubcoreMesh(core_axis_name="c", subcore_axis_name="s")
    f = functools.partial(
        pl.kernel,
        out_type=(jax.ShapeDtypeStruct((NC, N, D), jnp.float32),
                  jax.ShapeDtypeStruct((NW, N), jnp.float32)),
        mesh=mesh,
        compiler_params=pltpu.CompilerParams(needs_layout_passes=False),
        scratch_types=[
            pltpu.VMEM((C,), jnp.int32),
            pltpu.VMEM((C,), jnp.int32),
            pltpu.VMEM((C,), jnp.int32),
            pltpu.VMEM((C,), jnp.int32),
            pltpu.VMEM((C, D), jnp.float32),
            pltpu.VMEM((C, D), jnp.float32),
            pltpu.VMEM((C, D), jnp.float32),
            pltpu.VMEM((C, D), jnp.float32),
            pltpu.VMEM((C,), jnp.float32),
            pltpu.VMEM((C,), jnp.float32),
            pltpu.VMEM((16,), jnp.float32),
            pltpu.VMEM((N,), jnp.float32),
            pltpu.VMEM((16,), jnp.float32),
            pltpu.VMEM((16,), jnp.int32),
            pltpu.VMEM_SHARED((N, D), jnp.float32),
            pltpu.SemaphoreType.DMA,
            pltpu.SemaphoreType.DMA,
        ],
    )(_sc_edge_body)
    return f(hn, mult_flat, src, dst, beta16, zrows)


def kernel(x, edge_index, W1, b1, beta2):
    w1t = W1.T
    b1r = b1.reshape(1, D)
    beta11 = beta2.reshape(1, 1).astype(jnp.float32)
    src = edge_index[0]
    dst = edge_index[1]
    zrows = jnp.zeros((ZR, D), jnp.float32)
    beta1v = jnp.ones((16,), jnp.float32)
    beta2v = jnp.broadcast_to(beta2, (16,)).astype(jnp.float32)

    h1, hn1, mult1, selfw1 = _tc_prep(x, w1t, b1r)
    num1, den1 = _sc_edge(hn1, mult1.reshape(N), src, dst, beta1v, zrows)
    h2, hn2, mult2, selfw2 = _tc_combine(num1, den1.T, h1, selfw1, beta11)
    num2, den2 = _sc_edge(hn2, mult2.reshape(N), src, dst, beta2v, zrows)
    out, _, _, _ = _tc_combine(num2, den2.T, h2, selfw2, beta11)
    return out


# speedup vs baseline: 18.9374x; 1.0097x over previous
import functools

import jax
import jax.numpy as jnp
from jax import lax
from jax.experimental import pallas as pl
from jax.experimental.pallas import tpu as pltpu
from jax.experimental.pallas import tpu_sc as plsc

N = 10000
D = 128
E = 320000
NC = 2
NS = 16
NW = NC * NS
EPW = E // NW
C = 64
TAIL = 16
NFULL = (EPW - TAIL) // C
NPAIR = NFULL // 2
ZR = 2000
RB = 2000
EPS_NORM = 1e-12
EPS_DEN = 1e-16


def _prep_body(x_ref, w_ref, b_ref, h_ref, hn_ref, mult_ref, selfw_ref):
    xb = x_ref[...]
    h = jnp.maximum(
        lax.dot_general(xb, w_ref[...], (((1,), (0,)), ((), ())),
                        preferred_element_type=jnp.float32) + b_ref[...],
        0.0)
    ss = jnp.sum(h * h, axis=1, keepdims=True)
    cl = jnp.maximum(jnp.sqrt(ss), EPS_NORM)
    inv = 1.0 / cl
    h_ref[...] = h
    hn_ref[...] = h * inv
    mult_ref[...] = cl
    selfw_ref[...] = jnp.exp(ss * inv * inv)


def _tc_prep(x, w1t, b1r):
    return pl.pallas_call(
        _prep_body,
        grid=(N // RB,),
        in_specs=[
            pl.BlockSpec((RB, D), lambda i: (i, 0)),
            pl.BlockSpec((D, D), lambda i: (0, 0)),
            pl.BlockSpec((1, D), lambda i: (0, 0)),
        ],
        out_specs=[
            pl.BlockSpec((RB, D), lambda i: (i, 0)),
            pl.BlockSpec((RB, D), lambda i: (i, 0)),
            pl.BlockSpec((RB, 1), lambda i: (i, 0)),
            pl.BlockSpec((RB, 1), lambda i: (i, 0)),
        ],
        out_shape=[
            jax.ShapeDtypeStruct((N, D), jnp.float32),
            jax.ShapeDtypeStruct((N, D), jnp.float32),
            jax.ShapeDtypeStruct((N, 1), jnp.float32),
            jax.ShapeDtypeStruct((N, 1), jnp.float32),
        ],
    )(x, w1t, b1r)


def _combine_body(num_ref, den_ref, h_ref, selfw_ref, beta_ref,
                  o_ref, hn_ref, mult_ref, selfw2_ref):
    num = num_ref[0] + num_ref[1]
    den = jnp.sum(den_ref[...], axis=1, keepdims=True)
    sw = selfw_ref[...]
    h = h_ref[...]
    out = (num + sw * h) / (den + sw + EPS_DEN)
    ss = jnp.sum(out * out, axis=1, keepdims=True)
    cl = jnp.maximum(jnp.sqrt(ss), EPS_NORM)
    inv = 1.0 / cl
    o_ref[...] = out
    hn_ref[...] = out * inv
    mult_ref[...] = cl
    selfw2_ref[...] = jnp.exp(beta_ref[0, 0] * ss * inv * inv)


def _tc_combine(num, den_t, h, selfw, beta11):
    return pl.pallas_call(
        _combine_body,
        grid=(N // RB,),
        in_specs=[
            pl.BlockSpec((NC, RB, D), lambda i: (0, i, 0)),
            pl.BlockSpec((RB, NW), lambda i: (i, 0)),
            pl.BlockSpec((RB, D), lambda i: (i, 0)),
            pl.BlockSpec((RB, 1), lambda i: (i, 0)),
            pl.BlockSpec((1, 1), lambda i: (0, 0)),
        ],
        out_specs=[
            pl.BlockSpec((RB, D), lambda i: (i, 0)),
            pl.BlockSpec((RB, D), lambda i: (i, 0)),
            pl.BlockSpec((RB, 1), lambda i: (i, 0)),
            pl.BlockSpec((RB, 1), lambda i: (i, 0)),
        ],
        out_shape=[
            jax.ShapeDtypeStruct((N, D), jnp.float32),
            jax.ShapeDtypeStruct((N, D), jnp.float32),
            jax.ShapeDtypeStruct((N, 1), jnp.float32),
            jax.ShapeDtypeStruct((N, 1), jnp.float32),
        ],
    )(num, den_t, h, selfw, beta11)


def _sc_edge_body(hn_hbm, mult_hbm, src_hbm, dst_hbm, beta_hbm, zrows_hbm,
                  acc_hbm, den_hbm,
                  sidxA, didxA, sidxB, didxB, srowA, drowA, srowB, drowB,
                  multA, multB, betav, denv, wtmp, ktmp,
                  numsp, semA, semB):
    cid = lax.axis_index("c")
    sid = lax.axis_index("s")
    wid = sid * NC + cid
    lane = lax.iota(jnp.int32, 16)
    z16 = jnp.zeros((16,), jnp.float32)
    zi16 = jnp.zeros((16,), jnp.int32)
    wbase = wid * EPW

    pltpu.sync_copy(beta_hbm, betav)

    @pl.when(sid < N // ZR)
    def _zero_acc():
        pltpu.sync_copy(zrows_hbm, numsp.at[pl.ds(sid * ZR, ZR)])

    def zden(i, carry):
        denv[pl.ds(i * 16, 16)] = z16
        return carry
    lax.fori_loop(0, N // 16, zden, 0)
    plsc.subcore_barrier()
    beta = betav[...]

    def do_group(srow, drow, sidx, didx, multb, off):
        si = sidx[pl.ds(off, 16)]
        di = didx[pl.ds(off, 16)]
        mg = multb[pl.ds(off, 16)]
        wlo = z16
        whi = z16
        for e in range(16):
            sc = [srow[off + e, pl.ds(k * 16, 16)] for k in range(8)]
            acc = sc[0] * drow[off + e, pl.ds(0, 16)]
            for k in range(1, 8):
                acc = acc + sc[k] * drow[off + e, pl.ds(k * 16, 16)]
            dot = jnp.sum(acc)
            wev = jnp.exp(beta * dot)
            if e < 8:
                wlo = jnp.where(lane == e, wev, wlo)
            else:
                whi = jnp.where(lane == e, wev, whi)
            wse = jnp.where(si[e] == di[e], 0.0, wev[0] * mg[e])
            for k in range(8):
                drow[off + e, pl.ds(k * 16, 16)] = sc[k] * wse
        w = wlo + whi
        w = jnp.where(si == di, 0.0, w)
        sk, sw = plsc.sort_key_val(di, w)
        csum = plsc.cumsum(sw)
        ktmp[...] = sk
        key_next = plsc.load_gather(ktmp, [jnp.minimum(lane + 1, 15)])
        key_prev = plsc.load_gather(ktmp, [jnp.maximum(lane - 1, 0)])
        is_end = (lane == 15) | (sk != key_next)
        is_start = (lane == 0) | (sk != key_prev)
        sstart = plsc.cummax(jnp.where(is_start, lane, 0))
        wtmp[...] = csum
        seg_base = jnp.where(
            sstart == 0, 0.0,
            plsc.load_gather(wtmp, [jnp.maximum(sstart - 1, 0)]))
        plsc.addupdate_scatter(denv, [sk], csum - seg_base, mask=is_end)

    def load_idx_and_fire(base, sidx, didx, multb, srow, drow, sem):
        pltpu.sync_copy(src_hbm.at[pl.ds(base, C)], sidx)
        pltpu.sync_copy(dst_hbm.at[pl.ds(base, C)], didx)
        pltpu.async_copy(hn_hbm.at[sidx], srow, sem)
        pltpu.async_copy(hn_hbm.at[didx], drow, sem)
        pltpu.async_copy(mult_hbm.at[sidx], multb, sem)

    def wait_set(srow, drow, multb, sem):
        pltpu.make_async_copy(hn_hbm.at[pl.ds(0, C)], srow, sem).wait()
        pltpu.make_async_copy(hn_hbm.at[pl.ds(0, C)], drow, sem).wait()
        pltpu.make_async_copy(mult_hbm.at[pl.ds(0, C)], multb, sem).wait()

    def compute_chunk(sidx, didx, srow, drow, multb):
        def grp(g, carry):
            do_group(srow, drow, sidx, didx, multb, g * 16)
            return carry
        lax.fori_loop(0, C // 16, grp, 0)
        pltpu.sync_copy(drow, numsp.at[didx], add=True)

    load_idx_and_fire(wbase, sidxA, didxA, multA, srowA, drowA, semA)
    wait_set(srowA, drowA, multA, semA)
    do_group(srowA, drowA, sidxA, didxA, multA, 0)
    for r in range(TAIL, C):
        for k in range(8):
            drowA[r, pl.ds(k * 16, 16)] = z16
    for r in range(TAIL, C, 16):
        didxA[pl.ds(r, 16)] = zi16
    pltpu.sync_copy(drowA, numsp.at[didxA], add=True)

    load_idx_and_fire(wbase + TAIL, sidxA, didxA, multA, srowA, drowA, semA)

    def pair(i, carry):
        wait_set(srowA, drowA, multA, semA)
        load_idx_and_fire(wbase + TAIL + (2 * i + 1) * C,
                          sidxB, didxB, multB, srowB, drowB, semB)
        compute_chunk(sidxA, didxA, srowA, drowA, multA)

        @pl.when(i < NPAIR - 1)
        def _prefetch_a():
            load_idx_and_fire(wbase + TAIL + (2 * i + 2) * C,
                              sidxA, didxA, multA, srowA, drowA, semA)

        wait_set(srowB, drowB, multB, semB)
        compute_chunk(sidxB, didxB, srowB, drowB, multB)
        return carry

    lax.fori_loop(0, NPAIR, pair, 0)

    plsc.subcore_barrier()

    @pl.when(sid < N // ZR)
    def _write_acc():
        pltpu.sync_copy(numsp.at[pl.ds(sid * ZR, ZR)],
                        acc_hbm.at[cid, pl.ds(sid * ZR, ZR)])

    pltpu.sync_copy(denv, den_hbm.at[wid])


def _sc_edge(hn, mult_flat, src, dst, beta16, zrows):
    mesh = plsc.VectorSubcoreMesh(core_axis_name="c", subcore_axis_name="s")
    f = functools.partial(
        pl.kernel,
        out_type=(jax.ShapeDtypeStruct((NC, N, D), jnp.float32),
                  jax.ShapeDtypeStruct((NW, N), jnp.float32)),
        mesh=mesh,
        compiler_params=pltpu.CompilerParams(needs_layout_passes=False),
        scratch_types=[
            pltpu.VMEM((C,), jnp.int32),
            pltpu.VMEM((C,), jnp.int32),
            pltpu.VMEM((C,), jnp.int32),
            pltpu.VMEM((C,), jnp.int32),
            pltpu.VMEM((C, D), jnp.float32),
            pltpu.VMEM((C, D), jnp.float32),
            pltpu.VMEM((C, D), jnp.float32),
            pltpu.VMEM((C, D), jnp.float32),
            pltpu.VMEM((C,), jnp.float32),
            pltpu.VMEM((C,), jnp.float32),
            pltpu.VMEM((16,), jnp.float32),
            pltpu.VMEM((N,), jnp.float32),
            pltpu.VMEM((16,), jnp.float32),
            pltpu.VMEM((16,), jnp.int32),
            pltpu.VMEM_SHARED((N, D), jnp.float32),
            pltpu.SemaphoreType.DMA,
            pltpu.SemaphoreType.DMA,
        ],
    )(_sc_edge_body)
    return f(hn, mult_flat, src, dst, beta16, zrows)


def kernel(x, edge_index, W1, b1, beta2):
    w1t = W1.T
    b1r = b1.reshape(1, D)
    beta11 = beta2.reshape(1, 1).astype(jnp.float32)
    src = edge_index[0]
    dst = edge_index[1]
    zrows = jnp.zeros((ZR, D), jnp.float32)
    beta1v = jnp.ones((16,), jnp.float32)
    beta2v = jnp.broadcast_to(beta2, (16,)).astype(jnp.float32)

    h1, hn1, mult1, selfw1 = _tc_prep(x, w1t, b1r)
    num1, den1 = _sc_edge(hn1, mult1.reshape(N), src, dst, beta1v, zrows)
    h2, hn2, mult2, selfw2 = _tc_combine(num1, den1.T, h1, selfw1, beta11)
    num2, den2 = _sc_edge(hn2, mult2.reshape(N), src, dst, beta2v, zrows)
    out, _, _, _ = _tc_combine(num2, den2.T, h2, selfw2, beta11)
    return out
